# baseline (speedup 1.0000x reference)
import jax
import jax.numpy as jnp
from jax.experimental import pallas as pl
from jax.experimental.pallas import tpu as pltpu

_CT = (((1,), (0,)), ((), ()))
_CT_T = (((1,), (1,)), ((), ()))


def _make_body(n, h, d, cw1, nb1, nch1, cwn, nbn, ncn, cwa, nba, nca):
    def body(a_hbm, p1_hbm, p2_hbm, l1_hbm, l2_hbm, ego_ref,
             w1_ref, b1_ref, w2_ref, b2_ref, out_ref,
             ring1, ring_n, ring_a, p_scr, l_scr, acc,
             sem1, sem_n, sem_a):
        ego = ego_ref[...]

        def s1_phase(src_hbm, dst_scr):
            def start(j, b):
                pltpu.make_async_copy(
                    src_hbm.at[j], ring1.at[b], sem1.at[b]).start()

            for b in range(nb1):
                start(b, b)

            def rnd(r, carry):
                for b in range(nb1):
                    j = r * nb1 + b
                    pltpu.make_async_copy(
                        src_hbm.at[0], ring1.at[b], sem1.at[b]).wait()
                    dst_scr[pl.ds(j * cw1, cw1), :] = jax.lax.dot_general(
                        ring1[b], ego, _CT,
                        preferred_element_type=jnp.float32)

                    def nxt():
                        start(j + nb1, b)
                    pl.when(j + nb1 < nch1)(nxt)
                return carry

            jax.lax.fori_loop(0, nch1 // nb1, rnd, 0, unroll=False)

        s1_phase(p1_hbm, p_scr)
        s1_phase(l1_hbm, l_scr)

        def s2_phase(src_hbm, rhs_scr, first):
            def start(j, b):
                pltpu.make_async_copy(
                    src_hbm.at[pl.ds(j * cwn, cwn), :], ring_n.at[b],
                    sem_n.at[b]).start()

            for b in range(nbn):
                start(b, b)

            rhs = rhs_scr[...]

            def rnd(r, carry):
                for b in range(nbn):
                    j = r * nbn + b
                    pltpu.make_async_copy(
                        src_hbm.at[pl.ds(0, cwn), :], ring_n.at[b],
                        sem_n.at[b]).wait()
                    blk = jax.lax.dot_general(
                        ring_n[b], rhs, _CT,
                        preferred_element_type=jnp.float32)
                    if first:
                        acc[pl.ds(j * cwn, cwn), :] = blk
                    else:
                        acc[pl.ds(j * cwn, cwn), :] = (
                            acc[pl.ds(j * cwn, cwn), :] + blk)

                    def nxt():
                        start(j + nbn, b)
                    pl.when(j + nbn < ncn)(nxt)
                return carry

            jax.lax.fori_loop(0, ncn // nbn, rnd, 0, unroll=False)

        s2_phase(p2_hbm, p_scr, True)
        s2_phase(l2_hbm, l_scr, False)

        w1 = w1_ref[...]
        w2 = w2_ref[...]
        b1v = b1_ref[...]
        b2v = b2_ref[...]

        def a_start(i, b):
            pltpu.make_async_copy(
                a_hbm.at[i], ring_a.at[b], sem_a.at[b]).start()

        for b in range(nba):
            a_start(b, b)

        def a_rnd(r, carry):
            for b in range(nba):
                i = r * nba + b
                pltpu.make_async_copy(
                    a_hbm.at[0], ring_a.at[b], sem_a.at[b]).wait()
                side = jax.lax.dot_general(
                    ring_a[b], ego, _CT, preferred_element_type=jnp.float32)
                side = side + acc[pl.ds(i * cwa, cwa), :]

                def nxt():
                    a_start(i + nba, b)
                pl.when(i + nba < nca)(nxt)

                eg = ego_ref[pl.ds(i * cwa, cwa), :]
                s = jax.lax.dot_general(
                    eg + side, w1, _CT_T,
                    preferred_element_type=jnp.float32) + b1v
                t = jax.lax.dot_general(
                    eg * side, w2, _CT_T,
                    preferred_element_type=jnp.float32) + b2v
                s = jnp.where(s >= 0, s, 0.01 * s)
                t = jnp.where(t >= 0, t, 0.01 * t)
                out_ref[pl.ds(i * cwa, cwa), :] = s + t
            return carry

        jax.lax.fori_loop(0, nca // nba, a_rnd, 0, unroll=False)

    return body


def kernel(ego_embeddings, A_in, norm_proj1, norm_proj2, norm_lib1,
           norm_lib2, W1, b1, W2, b2, interpret=False):
    n, d = ego_embeddings.shape
    h = norm_proj1.shape[0]

    def pick(dim, target, cands):
        cw = target if dim % target == 0 else dim
        nc = dim // cw
        for c in cands:
            if nc % c == 0:
                return cw, c, nc
        return cw, 1, nc

    cw1, nb1, nch1 = pick(h, 64, (4, 2))
    cwn, nbn, ncn = pick(n, 200, (5, 4, 2))
    cwa, nba, nca = pick(n, 80, (5, 4, 2))

    a3 = A_in.reshape(nca, cwa, n)
    p1_3 = norm_proj1.reshape(nch1, cw1, n)
    l1_3 = norm_lib1.reshape(nch1, cw1, n)

    body = _make_body(n, h, d, cw1, nb1, nch1, cwn, nbn, ncn, cwa, nba, nca)

    out = pl.pallas_call(
        body,
        in_specs=[
            pl.BlockSpec(memory_space=pltpu.MemorySpace.HBM),
            pl.BlockSpec(memory_space=pltpu.MemorySpace.HBM),
            pl.BlockSpec(memory_space=pltpu.MemorySpace.HBM),
            pl.BlockSpec(memory_space=pltpu.MemorySpace.HBM),
            pl.BlockSpec(memory_space=pltpu.MemorySpace.HBM),
            pl.BlockSpec(memory_space=pltpu.MemorySpace.VMEM),
            pl.BlockSpec(memory_space=pltpu.MemorySpace.VMEM),
            pl.BlockSpec(memory_space=pltpu.MemorySpace.VMEM),
            pl.BlockSpec(memory_space=pltpu.MemorySpace.VMEM),
            pl.BlockSpec(memory_space=pltpu.MemorySpace.VMEM),
        ],
        out_specs=pl.BlockSpec(memory_space=pltpu.MemorySpace.VMEM),
        out_shape=jax.ShapeDtypeStruct((n, d), jnp.float32),
        scratch_shapes=[
            pltpu.VMEM((nb1, cw1, n), jnp.float32),
            pltpu.VMEM((nbn, cwn, h), jnp.float32),
            pltpu.VMEM((nba, cwa, n), jnp.float32),
            pltpu.VMEM((h, d), jnp.float32),
            pltpu.VMEM((h, d), jnp.float32),
            pltpu.VMEM((n, d), jnp.float32),
            pltpu.SemaphoreType.DMA((nb1,)),
            pltpu.SemaphoreType.DMA((nbn,)),
            pltpu.SemaphoreType.DMA((nba,)),
        ],
        compiler_params=pltpu.CompilerParams(
            vmem_limit_bytes=100 * 1024 * 1024),
        interpret=interpret,
    )(a3, p1_3, norm_proj2, l1_3, norm_lib2,
      ego_embeddings, W1, b1.reshape(1, d), W2, b2.reshape(1, d))
    return out
